# baseline (speedup 1.0000x reference)
import functools

import jax
import jax.numpy as jnp
from jax.experimental import pallas as pl
from jax.experimental.pallas import tpu as pltpu
from jax.experimental.pallas import tpu_sc as plsc

K = 8
E = 64
L = 16
NW = 32


def _gate_body(x_ref, w_ref, b_ref, o_ref):
    o_ref[0] = jax.lax.dot_general(
        x_ref[0], w_ref[...], (((0,), (1,)), ((), ())),
        preferred_element_type=jnp.float32,
    ) + b_ref[...]


def _gate_logits(x, W, b):
    B, C, H, Wd = x.shape
    T = H * Wd
    xr = x.reshape(B, C, T)
    b2 = b.reshape(1, E)
    logits = pl.pallas_call(
        _gate_body,
        grid=(B,),
        in_specs=[
            pl.BlockSpec((1, C, T), lambda i: (i, 0, 0)),
            pl.BlockSpec((E, C), lambda i: (0, 0)),
            pl.BlockSpec((1, E), lambda i: (0, 0)),
        ],
        out_specs=pl.BlockSpec((1, T, E), lambda i: (i, 0, 0)),
        out_shape=jax.ShapeDtypeStruct((B, T, E), jnp.float32),
    )(xr, W, b2)
    return logits.reshape(B * T // L, L, E)


def _lane_iota():
    return jax.lax.broadcasted_iota(jnp.int32, (L,), 0)


def _permute(v, idx):
    return jax.lax.gather(
        v, idx[:, None],
        jax.lax.GatherDimensionNumbers(
            offset_dims=(), collapsed_slice_dims=(0,), start_index_map=(0,)),
        (1,),
        mode=jax.lax.GatherScatterMode.PROMISE_IN_BOUNDS,
    )


def _merge_top8(a, b):
    ak, av = a
    bk, bv = b
    lanes = _lane_iota()
    in_a = lanes < 8
    sh = jnp.where(in_a, 0, lanes - 8)
    ck = jnp.where(in_a, ak, _permute(bk, sh))
    cv = jnp.where(in_a, av, _permute(bv, sh))
    return plsc.sort_key_val(ck, cv, descending=True)


def _topk_group(in_ref, wb_ref):
    lanes = _lane_iota()
    zero_idx = jnp.zeros((L,), jnp.int32)
    last_idx = jnp.full((L,), L - 1, jnp.int32)
    topk_mask = lanes < K
    sh = jnp.where(topk_mask, 0, lanes - K)
    for l in range(L):
        chunks = []
        for c in range(E // L):
            v = in_ref[l, pl.ds(c * L, L)]
            ids = lanes + (c * L)
            chunks.append(plsc.sort_key_val(v, ids, descending=True))
        m01 = _merge_top8(chunks[0], chunks[1])
        m23 = _merge_top8(chunks[2], chunks[3])
        mk, mv = _merge_top8(m01, m23)
        mx = _permute(mk, zero_idx)
        ez = jnp.exp(mk - mx)
        ez = jnp.where(topk_mask, ez, jnp.float32(0.0))
        tot = _permute(plsc.cumsum(ez), last_idx)
        mvf = plsc.bitcast(mv, jnp.float32)
        wb_ref[l] = jnp.where(topk_mask, ez / tot, _permute(mvf, sh))


def _topk_sc(logits):
    G = logits.shape[0]
    per = G // NW
    mesh = plsc.VectorSubcoreMesh(
        core_axis_name="c", subcore_axis_name="s", num_cores=2, num_subcores=16)

    @functools.partial(
        pl.kernel,
        out_type=jax.ShapeDtypeStruct((G, L, L), jnp.float32),
        mesh=mesh,
        compiler_params=pltpu.CompilerParams(needs_layout_passes=False),
        scratch_types=[
            pltpu.VMEM((L, E), jnp.float32),
            pltpu.VMEM((L, E), jnp.float32),
            pltpu.VMEM((L, L), jnp.float32),
            pltpu.VMEM((L, L), jnp.float32),
            pltpu.SemaphoreType.DMA((2,)),
            pltpu.SemaphoreType.DMA((2,)),
        ],
    )
    def run(lg_hbm, ow_hbm, in0, in1, wb0, wb1, isem, osem):
        wid = jax.lax.axis_index("s") * 2 + jax.lax.axis_index("c")
        base = wid * per

        def in_copy(g, buf, slot):
            return pltpu.make_async_copy(lg_hbm.at[g], buf, isem.at[slot])

        def out_copies(g, wb, so):
            return (
                pltpu.make_async_copy(wb, ow_hbm.at[g], osem.at[so]),
            )

        in_copy(base, in0, 0).start()

        def pair(p, carry):
            g0 = base + 2 * p
            g1 = g0 + 1
            in_copy(g1, in1, 1).start()
            in_copy(g0, in0, 0).wait()

            @pl.when(p > 0)
            def _():
                for cp in out_copies(g0 - 2, wb0, 0) + out_copies(
                        g1 - 2, wb1, 1):
                    cp.wait()

            _topk_group(in0, wb0)
            for cp in out_copies(g0, wb0, 0):
                cp.start()

            @pl.when(p + 1 < per // 2)
            def _():
                in_copy(g0 + 2, in0, 0).start()

            in_copy(g1, in1, 1).wait()
            _topk_group(in1, wb1)
            for cp in out_copies(g1, wb1, 1):
                cp.start()
            return carry

        jax.lax.fori_loop(0, per // 2, pair, 0)
        for cp in out_copies(base, wb0, 0) + out_copies(base, wb1, 1):
            cp.wait()

    return run(logits)


def kernel(x, W, b):
    B, C, H, Wd = x.shape
    logits = _gate_logits(x, W, b)
    packed = _topk_sc(logits)
    w = packed[:, :, :K].reshape(B, H, Wd, K)
    i = jax.lax.bitcast_convert_type(
        packed[:, :, K:], jnp.int32).reshape(B, H, Wd, K)
    return w, i


# speedup vs baseline: 1.0581x; 1.0581x over previous
import functools

import jax
import jax.numpy as jnp
from jax.experimental import pallas as pl
from jax.experimental.pallas import tpu as pltpu
from jax.experimental.pallas import tpu_sc as plsc

K = 8
E = 64
L = 16
NW = 32


def _gate_body(x_ref, w_ref, b_ref, o_ref):
    o_ref[0] = jax.lax.dot_general(
        x_ref[0], w_ref[...], (((0,), (1,)), ((), ())),
        preferred_element_type=jnp.float32,
    ) + b_ref[...]


def _gate_logits(x, W, b):
    B, C, H, Wd = x.shape
    T = H * Wd
    xr = x.reshape(B, C, T)
    b2 = b.reshape(1, E)
    logits = pl.pallas_call(
        _gate_body,
        grid=(B,),
        in_specs=[
            pl.BlockSpec((1, C, T), lambda i: (i, 0, 0)),
            pl.BlockSpec((E, C), lambda i: (0, 0)),
            pl.BlockSpec((1, E), lambda i: (0, 0)),
        ],
        out_specs=pl.BlockSpec((1, T, E), lambda i: (i, 0, 0)),
        out_shape=jax.ShapeDtypeStruct((B, T, E), jnp.float32),
        compiler_params=pltpu.CompilerParams(
            allow_input_fusion=[True, False, False]),
    )(xr, W, b2)
    return logits.reshape(B * T // L, L, E)


def _lane_iota():
    return jax.lax.broadcasted_iota(jnp.int32, (L,), 0)


def _permute(v, idx):
    return jax.lax.gather(
        v, idx[:, None],
        jax.lax.GatherDimensionNumbers(
            offset_dims=(), collapsed_slice_dims=(0,), start_index_map=(0,)),
        (1,),
        mode=jax.lax.GatherScatterMode.PROMISE_IN_BOUNDS,
    )


def _merge_top8(a, b):
    ak, av = a
    bk, bv = b
    lanes = _lane_iota()
    in_a = lanes < 8
    sh = jnp.where(in_a, 0, lanes - 8)
    ck = jnp.where(in_a, ak, _permute(bk, sh))
    cv = jnp.where(in_a, av, _permute(bv, sh))
    return plsc.sort_key_val(ck, cv, descending=True)


def _topk_group(in_ref, wb_ref, ib_ref):
    lanes = _lane_iota()
    zero_idx = jnp.zeros((L,), jnp.int32)
    last_idx = jnp.full((L,), L - 1, jnp.int32)
    topk_mask = lanes < K
    for l in range(L):
        chunks = []
        for c in range(E // L):
            v = in_ref[l, pl.ds(c * L, L)]
            ids = lanes + (c * L)
            chunks.append(plsc.sort_key_val(v, ids, descending=True))
        m01 = _merge_top8(chunks[0], chunks[1])
        m23 = _merge_top8(chunks[2], chunks[3])
        mk, mv = _merge_top8(m01, m23)
        mx = _permute(mk, zero_idx)
        ez = jnp.exp(mk - mx)
        ez = jnp.where(topk_mask, ez, jnp.float32(0.0))
        tot = _permute(plsc.cumsum(ez), last_idx)
        wb_ref[l] = ez / tot
        ib_ref[l] = mv


def _topk_sc(logits):
    G = logits.shape[0]
    per = G // NW
    mesh = plsc.VectorSubcoreMesh(
        core_axis_name="c", subcore_axis_name="s", num_cores=2, num_subcores=16)

    @functools.partial(
        pl.kernel,
        out_type=[
            jax.ShapeDtypeStruct((G, L, L), jnp.float32),
            jax.ShapeDtypeStruct((G, L, L), jnp.int32),
        ],
        mesh=mesh,
        compiler_params=pltpu.CompilerParams(needs_layout_passes=False),
        scratch_types=[
            pltpu.VMEM((L, E), jnp.float32),
            pltpu.VMEM((L, E), jnp.float32),
            pltpu.VMEM((L, L), jnp.float32),
            pltpu.VMEM((L, L), jnp.int32),
            pltpu.VMEM((L, L), jnp.float32),
            pltpu.VMEM((L, L), jnp.int32),
            pltpu.SemaphoreType.DMA((2,)),
            pltpu.SemaphoreType.DMA((4,)),
        ],
    )
    def run(lg_hbm, ow_hbm, oi_hbm, in0, in1, wb0, ib0, wb1, ib1, isem, osem):
        wid = jax.lax.axis_index("s") * 2 + jax.lax.axis_index("c")
        base = wid * per

        def in_copy(g, buf, slot):
            return pltpu.make_async_copy(lg_hbm.at[g], buf, isem.at[slot])

        def out_copies(g, wb, ib, so):
            return (
                pltpu.make_async_copy(wb, ow_hbm.at[g], osem.at[so]),
                pltpu.make_async_copy(ib, oi_hbm.at[g], osem.at[so + 1]),
            )

        in_copy(base, in0, 0).start()

        def pair(p, carry):
            g0 = base + 2 * p
            g1 = g0 + 1
            in_copy(g1, in1, 1).start()
            in_copy(g0, in0, 0).wait()

            @pl.when(p > 0)
            def _():
                for cp in out_copies(g0 - 2, wb0, ib0, 0) + out_copies(
                        g1 - 2, wb1, ib1, 2):
                    cp.wait()

            _topk_group(in0, wb0, ib0)
            for cp in out_copies(g0, wb0, ib0, 0):
                cp.start()

            @pl.when(p + 1 < per // 2)
            def _():
                in_copy(g0 + 2, in0, 0).start()

            in_copy(g1, in1, 1).wait()
            _topk_group(in1, wb1, ib1)
            for cp in out_copies(g1, wb1, ib1, 2):
                cp.start()
            return carry

        jax.lax.fori_loop(0, per // 2, pair, 0)
        for cp in out_copies(base, wb0, ib0, 0) + out_copies(base, wb1, ib1, 2):
            cp.wait()

    return run(logits)


def kernel(x, W, b):
    B, C, H, Wd = x.shape
    logits = _gate_logits(x, W, b)
    w16, i16 = _topk_sc(logits)
    return (w16[:, :, :K].reshape(B, H, Wd, K),
            i16[:, :, :K].reshape(B, H, Wd, K))


# speedup vs baseline: 1.0628x; 1.0044x over previous
import functools

import jax
import jax.numpy as jnp
from jax.experimental import pallas as pl
from jax.experimental.pallas import tpu as pltpu
from jax.experimental.pallas import tpu_sc as plsc

K = 8
E = 64
L = 16
NW = 32


def _gate_body(x_ref, w_ref, b_ref, o_ref):
    o_ref[0] = jax.lax.dot_general(
        x_ref[0], w_ref[...], (((0,), (1,)), ((), ())),
        preferred_element_type=jnp.float32,
    ) + b_ref[...]


def _gate_logits(x, W, b):
    B, C, H, Wd = x.shape
    T = H * Wd
    xr = x.reshape(B, C, T)
    b2 = b.reshape(1, E)
    logits = pl.pallas_call(
        _gate_body,
        grid=(B,),
        in_specs=[
            pl.BlockSpec((1, C, T), lambda i: (i, 0, 0)),
            pl.BlockSpec((E, C), lambda i: (0, 0)),
            pl.BlockSpec((1, E), lambda i: (0, 0)),
        ],
        out_specs=pl.BlockSpec((1, T, E), lambda i: (i, 0, 0)),
        out_shape=jax.ShapeDtypeStruct((B, T, E), jnp.float32),
    )(xr, W, b2)
    return logits.reshape(B * T // L, L, E)


def _lane_iota():
    return jax.lax.broadcasted_iota(jnp.int32, (L,), 0)


def _permute(v, idx):
    return jax.lax.gather(
        v, idx[:, None],
        jax.lax.GatherDimensionNumbers(
            offset_dims=(), collapsed_slice_dims=(0,), start_index_map=(0,)),
        (1,),
        mode=jax.lax.GatherScatterMode.PROMISE_IN_BOUNDS,
    )


def _merge_top8(a, b, *, descending):
    ak, av = a
    bk, bv = b
    lo = _lane_iota() < 8
    if descending:
        ck = jnp.where(lo, ak, bk)
        cv = jnp.where(lo, av, bv)
    else:
        ck = jnp.where(lo, bk, ak)
        cv = jnp.where(lo, bv, av)
    return plsc.sort_key_val(ck, cv, descending=descending)


def _topk_group(in_ref, wb_ref, ib_ref):
    lanes = _lane_iota()
    last_idx = jnp.full((L,), L - 1, jnp.int32)
    topk_mask = lanes < K
    descs = (True, False, False, True)
    for l in range(L):
        chunks = []
        for c in range(E // L):
            v = in_ref[l, pl.ds(c * L, L)]
            ids = lanes + (c * L)
            chunks.append(plsc.sort_key_val(v, ids, descending=descs[c]))
        m01 = _merge_top8(chunks[0], chunks[1], descending=True)
        m23 = _merge_top8(chunks[2], chunks[3], descending=False)
        mk, mv = _merge_top8(m01, m23, descending=True)
        ez = jnp.where(topk_mask, jnp.exp(mk), jnp.float32(0.0))
        tot = _permute(plsc.cumsum(ez), last_idx)
        wb_ref[l] = ez / tot
        ib_ref[l] = mv


def _topk_sc(logits):
    G = logits.shape[0]
    per = G // NW
    mesh = plsc.VectorSubcoreMesh(
        core_axis_name="c", subcore_axis_name="s", num_cores=2, num_subcores=16)

    @functools.partial(
        pl.kernel,
        out_type=[
            jax.ShapeDtypeStruct((G, L, L), jnp.float32),
            jax.ShapeDtypeStruct((G, L, L), jnp.int32),
        ],
        mesh=mesh,
        compiler_params=pltpu.CompilerParams(needs_layout_passes=False),
        scratch_types=[
            pltpu.VMEM((L, E), jnp.float32),
            pltpu.VMEM((L, E), jnp.float32),
            pltpu.VMEM((L, L), jnp.float32),
            pltpu.VMEM((L, L), jnp.int32),
            pltpu.VMEM((L, L), jnp.float32),
            pltpu.VMEM((L, L), jnp.int32),
            pltpu.SemaphoreType.DMA((2,)),
            pltpu.SemaphoreType.DMA((4,)),
        ],
    )
    def run(lg_hbm, ow_hbm, oi_hbm, in0, in1, wb0, ib0, wb1, ib1, isem, osem):
        wid = jax.lax.axis_index("s") * 2 + jax.lax.axis_index("c")
        base = wid * per

        def in_copy(g, buf, slot):
            return pltpu.make_async_copy(lg_hbm.at[g], buf, isem.at[slot])

        def out_copies(g, wb, ib, so):
            return (
                pltpu.make_async_copy(wb, ow_hbm.at[g], osem.at[so]),
                pltpu.make_async_copy(ib, oi_hbm.at[g], osem.at[so + 1]),
            )

        in_copy(base, in0, 0).start()

        def pair(p, carry):
            g0 = base + 2 * p
            g1 = g0 + 1
            in_copy(g1, in1, 1).start()
            in_copy(g0, in0, 0).wait()

            @pl.when(p > 0)
            def _():
                for cp in out_copies(g0 - 2, wb0, ib0, 0) + out_copies(
                        g1 - 2, wb1, ib1, 2):
                    cp.wait()

            _topk_group(in0, wb0, ib0)
            for cp in out_copies(g0, wb0, ib0, 0):
                cp.start()

            @pl.when(p + 1 < per // 2)
            def _():
                in_copy(g0 + 2, in0, 0).start()

            in_copy(g1, in1, 1).wait()
            _topk_group(in1, wb1, ib1)
            for cp in out_copies(g1, wb1, ib1, 2):
                cp.start()
            return carry

        jax.lax.fori_loop(0, per // 2, pair, 0)
        for cp in out_copies(base, wb0, ib0, 0) + out_copies(base, wb1, ib1, 2):
            cp.wait()

    return run(logits)


def kernel(x, W, b):
    B, C, H, Wd = x.shape
    logits = _gate_logits(x, W, b)
    w16, i16 = _topk_sc(logits)
    return (w16[:, :, :K].reshape(B, H, Wd, K),
            i16[:, :, :K].reshape(B, H, Wd, K))
